# Initial kernel scaffold; baseline (speedup 1.0000x reference)
import jax
import jax.numpy as jnp
from jax.experimental import pallas as pl


def kernel(x, edge_index1, edge_weight1, edge_index2, edge_weight2, edge_index3, edge_weight3, W0, b0, W1a, b1a, W1b, b1b, Ws1, bs1, W2a, b2a, W2b, b2b, Ws2, bs2, W3a, b3a, W3b, b3b, Ws3, bs3, Wfc, bfc):
    raise NotImplementedError("write your pallas kernel here")


# speedup vs baseline: 30.1015x; 30.1015x over previous
import functools

import jax
import jax.numpy as jnp
from jax import lax
from jax.experimental import pallas as pl
from jax.experimental.pallas import tpu as pltpu
from jax.experimental.pallas import tpu_sc as plsc

_NS = 16
_K = 128
_ZR = 8


@functools.lru_cache(maxsize=None)
def _make_lop(V, C, E):
    eps = E // _NS
    nch = eps // _K
    stripe = V // _NS
    nz = stripe // _ZR
    mesh = plsc.VectorSubcoreMesh(core_axis_name="c", subcore_axis_name="s")
    sds = jax.ShapeDtypeStruct((V, C), jnp.float32)

    @functools.partial(
        pl.kernel,
        mesh=mesh,
        compiler_params=pltpu.CompilerParams(needs_layout_passes=False,
                                             use_tc_tiling_on_sc=False),
        out_type=[sds, sds, sds, sds],
        scratch_types=[
            pltpu.VMEM_SHARED((V, C), jnp.float32),
            pltpu.VMEM((_K,), jnp.int32),
            pltpu.VMEM((_K,), jnp.int32),
            pltpu.VMEM((_K,), jnp.float32),
            pltpu.VMEM((_K, C), jnp.float32),
            pltpu.VMEM((_ZR, C), jnp.float32),
            pltpu.SemaphoreType.DMA,
        ],
    )
    def lop_k(x0, x1, x2, x3, src_h, dst_h, w_h, y0, y1, y2, y3,
              acc, src_v, dst_v, w_v, rows_v, zbuf, sem):
        cid = lax.axis_index("c")
        sid = lax.axis_index("s")
        iota = lax.iota(jnp.int32, 16)
        zvec = jnp.zeros((16,), jnp.float32)
        for r in range(_ZR):
            for i in range(C // 16):
                zbuf[r, pl.ds(i * 16, 16)] = zvec

        def phase(x_h, y_h):
            base = sid * stripe

            def zbody(i, c):
                pltpu.sync_copy(zbuf, acc.at[pl.ds(base + i * _ZR, _ZR)])
                return c

            lax.fori_loop(0, nz, zbody, 0)
            plsc.subcore_barrier()

            e0 = sid * eps

            def cbody(g, c):
                eb = e0 + g * _K
                pltpu.sync_copy(src_h.at[pl.ds(eb, _K)], src_v)
                pltpu.sync_copy(dst_h.at[pl.ds(eb, _K)], dst_v)
                pltpu.sync_copy(w_h.at[pl.ds(eb, _K)], w_v)
                pltpu.async_copy(x_h.at[src_v], rows_v, sem).wait()

                def ebody(k, c2):
                    ksp = jnp.full((16,), k, jnp.int32)
                    wsp = plsc.load_gather(w_v, [ksp])
                    for i in range(C // 16):
                        col = iota + (i * 16)
                        gv = plsc.load_gather(rows_v, [ksp, col])
                        plsc.store_scatter(rows_v, [ksp, col], gv * wsp)
                    return c2

                lax.fori_loop(0, _K, ebody, 0)
                pltpu.sync_copy(rows_v, acc.at[dst_v], add=True)
                return c

            lax.fori_loop(0, nch, cbody, 0)
            plsc.subcore_barrier()
            pltpu.sync_copy(acc.at[pl.ds(base, stripe)],
                            y_h.at[pl.ds(base, stripe)])
            plsc.subcore_barrier()

        @pl.when(cid == 0)
        def _():
            phase(x0, y0)
            phase(x1, y1)

        @pl.when(cid == 1)
        def _():
            phase(x2, y2)
            phase(x3, y3)

    return lop_k


def _lop4(lopk, t, src, dst, w):
    ys = lopk(t[0], t[1], t[2], t[3], src, dst, w)
    return jnp.stack(ys)


_BN = 512


def _axpy(s, u):
    B, V, C = s.shape
    n = B * V
    sf, uf = s.reshape(n, C), u.reshape(n, C)

    def kfn(sr, ur, outr):
        outr[...] = 2.0 * sr[...] - ur[...]

    out = pl.pallas_call(
        kfn,
        grid=(n // _BN,),
        in_specs=[pl.BlockSpec((_BN, C), lambda i: (i, 0))] * 2,
        out_specs=pl.BlockSpec((_BN, C), lambda i: (i, 0)),
        out_shape=jax.ShapeDtypeStruct((n, C), jnp.float32),
    )(sf, uf)
    return out.reshape(B, V, C)


def _max4(a, b, c, d):
    n, C = a.shape

    def kfn(ar, br, cr, dr, outr):
        outr[...] = jnp.maximum(jnp.maximum(ar[...], br[...]),
                                jnp.maximum(cr[...], dr[...]))

    return pl.pallas_call(
        kfn,
        grid=(n // _BN,),
        in_specs=[pl.BlockSpec((_BN, C), lambda i: (i, 0))] * 4,
        out_specs=pl.BlockSpec((_BN, C), lambda i: (i, 0)),
        out_shape=jax.ShapeDtypeStruct((n, C), jnp.float32),
    )(a, b, c, d)


def _pool(t, nx3, nx2, nx1):
    B, V, C = t.shape
    r = t.reshape(B, nx3, nx2 // 2, 2, nx1 // 2, 2, C)
    parts = [r[:, :, :, p2, :, p1, :].reshape(-1, C)
             for p2 in (0, 1) for p1 in (0, 1)]
    out = _max4(*parts)
    return out.reshape(B, V // 4, C)


def _combine(t0, s1, t2, s3, W, bias, relu, xs=None, Ws=None):
    B, V, C = t0.shape
    n = B * V
    O = W.shape[2]
    A0, A1, A2, A3 = W[0], W[1] - W[3], W[2], 2.0 * W[3]
    b2 = jnp.tile(bias[None, :], (8, 1))
    ins = [t0.reshape(n, C), s1.reshape(n, C), t2.reshape(n, C),
           s3.reshape(n, C)]
    specs = [pl.BlockSpec((_BN, C), lambda i: (i, 0))] * 4
    has_sc = xs is not None
    if has_sc:
        Cs = xs.shape[2]
        ins.append(xs.reshape(n, Cs))
        specs.append(pl.BlockSpec((_BN, Cs), lambda i: (i, 0)))
        ins += [A0, A1, A2, A3, Ws, b2]
        specs += [pl.BlockSpec((C, O), lambda i: (0, 0))] * 4 + \
                 [pl.BlockSpec((Cs, O), lambda i: (0, 0)),
                  pl.BlockSpec((8, O), lambda i: (0, 0))]
    else:
        ins += [A0, A1, A2, A3, b2]
        specs += [pl.BlockSpec((C, O), lambda i: (0, 0))] * 4 + \
                 [pl.BlockSpec((8, O), lambda i: (0, 0))]

    def kfn(*refs):
        if has_sc:
            t0r, s1r, t2r, s3r, xsr, a0, a1, a2, a3, ws, br, outr = refs
        else:
            t0r, s1r, t2r, s3r, a0, a1, a2, a3, br, outr = refs
        acc = jnp.dot(t0r[...], a0[...], preferred_element_type=jnp.float32)
        acc += jnp.dot(s1r[...], a1[...], preferred_element_type=jnp.float32)
        acc += jnp.dot(t2r[...], a2[...], preferred_element_type=jnp.float32)
        acc += jnp.dot(s3r[...], a3[...], preferred_element_type=jnp.float32)
        if has_sc:
            acc += jnp.dot(xsr[...], ws[...],
                           preferred_element_type=jnp.float32)
        acc += br[0:1, :]
        if relu:
            acc = jnp.maximum(acc, 0.0)
        outr[...] = acc

    out = pl.pallas_call(
        kfn,
        grid=(n // _BN,),
        in_specs=specs,
        out_specs=pl.BlockSpec((_BN, O), lambda i: (i, 0)),
        out_shape=jax.ShapeDtypeStruct((n, O), jnp.float32),
    )(*ins)
    return out.reshape(B, V, O)


def _head(t, Wfc, bfc):
    B, V, C = t.shape
    O = Wfc.shape[1]
    tf = t.reshape(B * V, C)
    b2 = jnp.tile(bfc[None, :], (8, 1))

    def kfn(tr, wr, br, outr):
        m = jnp.max(tr[...].reshape(B, V, C), axis=1)
        z = jnp.dot(m, wr[...], preferred_element_type=jnp.float32)
        z += br[0:1, :]
        zm = jnp.max(z, axis=1, keepdims=True)
        lse = jnp.log(jnp.sum(jnp.exp(z - zm), axis=1, keepdims=True)) + zm
        outr[...] = z - lse

    return pl.pallas_call(
        kfn,
        out_shape=jax.ShapeDtypeStruct((B, O), jnp.float32),
    )(tf, Wfc, b2)


def _cheb(t0, src, dst, w, lopk, W, bias, relu, xs=None, Ws=None):
    s1 = _lop4(lopk, t0, src, dst, w)
    s2 = _lop4(lopk, s1, src, dst, w)
    t2 = _axpy(s2, t0)
    s3 = _lop4(lopk, t2, src, dst, w)
    return _combine(t0, s1, t2, s3, W, bias, relu, xs=xs, Ws=Ws)


def _res(t, src, dst, w, lopa, lopb, Wa, ba, Wb, bb, Ws, bs):
    h = _cheb(t, src, dst, w, lopa, Wa, ba, True)
    return _cheb(h, src, dst, w, lopb, Wb, bb + bs, True, xs=t, Ws=Ws)


def kernel(x, edge_index1, edge_weight1, edge_index2, edge_weight2,
           edge_index3, edge_weight3, W0, b0, W1a, b1a, W1b, b1b, Ws1, bs1,
           W2a, b2a, W2b, b2b, Ws2, bs2, W3a, b3a, W3b, b3b, Ws3, bs3,
           Wfc, bfc):
    B, CIN, V1 = x.shape
    V2, V3 = V1 // 4, V1 // 16
    E1, E2, E3 = edge_index1.shape[1], edge_index2.shape[1], edge_index3.shape[1]
    nx3 = 6
    nx1_1 = 96
    nx1_2 = 48

    xb = jnp.transpose(x, (0, 2, 1))
    x16 = jnp.pad(xb, ((0, 0), (0, 0), (0, 16 - CIN)))
    W0p = jnp.pad(W0, ((0, 0), (0, 16 - CIN), (0, 0)))

    s1_, d1_, w1_ = edge_index1[0], edge_index1[1], edge_weight1
    s2_, d2_, w2_ = edge_index2[0], edge_index2[1], edge_weight2
    s3_, d3_, w3_ = edge_index3[0], edge_index3[1], edge_weight3

    H0, H1, H2, H3 = W0.shape[2], W1a.shape[2], W2a.shape[2], W3a.shape[2]
    lop1_16 = _make_lop(V1, 16, E1)
    lop1_32 = _make_lop(V1, H1, E1)
    lop2_32 = _make_lop(V2, H1, E2)
    lop2_64 = _make_lop(V2, H2, E2)
    lop3_64 = _make_lop(V3, H2, E3)
    lop3_128 = _make_lop(V3, H3, E3)

    out = _cheb(x16, s1_, d1_, w1_, lop1_16, W0p, b0, True)
    out = _res(out, s1_, d1_, w1_, lop1_16, lop1_32, W1a, b1a, W1b, b1b,
               Ws1, bs1)
    out = _pool(out, nx3, nx1_1, nx1_1)
    out = _res(out, s2_, d2_, w2_, lop2_32, lop2_64, W2a, b2a, W2b, b2b,
               Ws2, bs2)
    out = _pool(out, nx3, nx1_2, nx1_2)
    out = _res(out, s3_, d3_, w3_, lop3_64, lop3_128, W3a, b3a, W3b, b3b,
               Ws3, bs3)
    return _head(out, Wfc, bfc)


# speedup vs baseline: 38.0296x; 1.2634x over previous
import functools

import jax
import jax.numpy as jnp
from jax import lax
from jax.experimental import pallas as pl
from jax.experimental.pallas import tpu as pltpu
from jax.experimental.pallas import tpu_sc as plsc

_NS = 16
_K = 128
_ZR = 8


@functools.lru_cache(maxsize=None)
def _make_lop(V, C, E):
    eps = E // _NS
    nch = eps // _K
    stripe = V // _NS
    nz = stripe // _ZR
    mesh = plsc.VectorSubcoreMesh(core_axis_name="c", subcore_axis_name="s")
    sds = jax.ShapeDtypeStruct((V, C), jnp.float32)

    @functools.partial(
        pl.kernel,
        mesh=mesh,
        compiler_params=pltpu.CompilerParams(needs_layout_passes=False,
                                             use_tc_tiling_on_sc=False),
        out_type=[sds, sds, sds, sds],
        scratch_types=[
            pltpu.VMEM_SHARED((V, C), jnp.float32),
            pltpu.VMEM((_K,), jnp.int32),
            pltpu.VMEM((_K,), jnp.int32),
            pltpu.VMEM((_K,), jnp.float32),
            pltpu.VMEM((_K, C), jnp.float32),
            pltpu.VMEM((_ZR, C), jnp.float32),
            pltpu.SemaphoreType.DMA,
        ],
    )
    def lop_k(x0, x1, x2, x3, src_h, dst_h, w_h, y0, y1, y2, y3,
              acc, src_v, dst_v, w_v, rows_v, zbuf, sem):
        cid = lax.axis_index("c")
        sid = lax.axis_index("s")
        iota = lax.iota(jnp.int32, 16)
        zvec = jnp.zeros((16,), jnp.float32)
        for r in range(_ZR):
            for i in range(C // 16):
                zbuf[r, pl.ds(i * 16, 16)] = zvec

        def phase(x_h, y_h):
            base = sid * stripe

            def zbody(i, c):
                pltpu.sync_copy(zbuf, acc.at[pl.ds(base + i * _ZR, _ZR)])
                return c

            lax.fori_loop(0, nz, zbody, 0)
            plsc.subcore_barrier()

            e0 = sid * eps

            def cbody(g, c):
                eb = e0 + g * _K
                c1 = pltpu.async_copy(src_h.at[pl.ds(eb, _K)], src_v, sem)
                c2 = pltpu.async_copy(dst_h.at[pl.ds(eb, _K)], dst_v, sem)
                c3 = pltpu.async_copy(w_h.at[pl.ds(eb, _K)], w_v, sem)
                c1.wait()
                c2.wait()
                c3.wait()
                pltpu.async_copy(x_h.at[src_v], rows_v, sem).wait()

                def ebody(k, c2):
                    ksp = jnp.full((16,), k, jnp.int32)
                    wsp = plsc.load_gather(w_v, [ksp])
                    for i in range(C // 16):
                        col = iota + (i * 16)
                        gv = plsc.load_gather(rows_v, [ksp, col])
                        plsc.store_scatter(rows_v, [ksp, col], gv * wsp)
                    return c2

                lax.fori_loop(0, _K, ebody, 0)
                pltpu.sync_copy(rows_v, acc.at[dst_v], add=True)
                return c

            lax.fori_loop(0, nch, cbody, 0)
            plsc.subcore_barrier()
            pltpu.sync_copy(acc.at[pl.ds(base, stripe)],
                            y_h.at[pl.ds(base, stripe)])
            plsc.subcore_barrier()

        @pl.when(cid == 0)
        def _():
            phase(x0, y0)
            phase(x1, y1)

        @pl.when(cid == 1)
        def _():
            phase(x2, y2)
            phase(x3, y3)

    return lop_k


def _lop4(lopk, t, src, dst, w):
    ys = lopk(t[0], t[1], t[2], t[3], src, dst, w)
    return jnp.stack(ys)


_BN = 512


def _axpy(s, u):
    B, V, C = s.shape
    n = B * V
    sf, uf = s.reshape(n, C), u.reshape(n, C)

    def kfn(sr, ur, outr):
        outr[...] = 2.0 * sr[...] - ur[...]

    out = pl.pallas_call(
        kfn,
        grid=(n // _BN,),
        in_specs=[pl.BlockSpec((_BN, C), lambda i: (i, 0))] * 2,
        out_specs=pl.BlockSpec((_BN, C), lambda i: (i, 0)),
        out_shape=jax.ShapeDtypeStruct((n, C), jnp.float32),
    )(sf, uf)
    return out.reshape(B, V, C)


def _max4(a, b, c, d):
    n, C = a.shape

    def kfn(ar, br, cr, dr, outr):
        outr[...] = jnp.maximum(jnp.maximum(ar[...], br[...]),
                                jnp.maximum(cr[...], dr[...]))

    return pl.pallas_call(
        kfn,
        grid=(n // _BN,),
        in_specs=[pl.BlockSpec((_BN, C), lambda i: (i, 0))] * 4,
        out_specs=pl.BlockSpec((_BN, C), lambda i: (i, 0)),
        out_shape=jax.ShapeDtypeStruct((n, C), jnp.float32),
    )(a, b, c, d)


def _pool(t, nx3, nx2, nx1):
    B, V, C = t.shape
    r = t.reshape(B, nx3, nx2 // 2, 2, nx1 // 2, 2, C)
    parts = [r[:, :, :, p2, :, p1, :].reshape(-1, C)
             for p2 in (0, 1) for p1 in (0, 1)]
    out = _max4(*parts)
    return out.reshape(B, V // 4, C)


def _combine(t0, s1, t2, s3, W, bias, relu, xs=None, Ws=None):
    B, V, C = t0.shape
    n = B * V
    O = W.shape[2]
    A0, A1, A2, A3 = W[0], W[1] - W[3], W[2], 2.0 * W[3]
    b2 = jnp.tile(bias[None, :], (8, 1))
    ins = [t0.reshape(n, C), s1.reshape(n, C), t2.reshape(n, C),
           s3.reshape(n, C)]
    specs = [pl.BlockSpec((_BN, C), lambda i: (i, 0))] * 4
    has_sc = xs is not None
    if has_sc:
        Cs = xs.shape[2]
        ins.append(xs.reshape(n, Cs))
        specs.append(pl.BlockSpec((_BN, Cs), lambda i: (i, 0)))
        ins += [A0, A1, A2, A3, Ws, b2]
        specs += [pl.BlockSpec((C, O), lambda i: (0, 0))] * 4 + \
                 [pl.BlockSpec((Cs, O), lambda i: (0, 0)),
                  pl.BlockSpec((8, O), lambda i: (0, 0))]
    else:
        ins += [A0, A1, A2, A3, b2]
        specs += [pl.BlockSpec((C, O), lambda i: (0, 0))] * 4 + \
                 [pl.BlockSpec((8, O), lambda i: (0, 0))]

    def kfn(*refs):
        if has_sc:
            t0r, s1r, t2r, s3r, xsr, a0, a1, a2, a3, ws, br, outr = refs
        else:
            t0r, s1r, t2r, s3r, a0, a1, a2, a3, br, outr = refs
        acc = jnp.dot(t0r[...], a0[...], preferred_element_type=jnp.float32)
        acc += jnp.dot(s1r[...], a1[...], preferred_element_type=jnp.float32)
        acc += jnp.dot(t2r[...], a2[...], preferred_element_type=jnp.float32)
        acc += jnp.dot(s3r[...], a3[...], preferred_element_type=jnp.float32)
        if has_sc:
            acc += jnp.dot(xsr[...], ws[...],
                           preferred_element_type=jnp.float32)
        acc += br[0:1, :]
        if relu:
            acc = jnp.maximum(acc, 0.0)
        outr[...] = acc

    out = pl.pallas_call(
        kfn,
        grid=(n // _BN,),
        in_specs=specs,
        out_specs=pl.BlockSpec((_BN, O), lambda i: (i, 0)),
        out_shape=jax.ShapeDtypeStruct((n, O), jnp.float32),
    )(*ins)
    return out.reshape(B, V, O)


def _head(t, Wfc, bfc):
    B, V, C = t.shape
    O = Wfc.shape[1]
    tf = t.reshape(B * V, C)
    b2 = jnp.tile(bfc[None, :], (8, 1))

    def kfn(tr, wr, br, outr):
        m = jnp.max(tr[...].reshape(B, V, C), axis=1)
        z = jnp.dot(m, wr[...], preferred_element_type=jnp.float32)
        z += br[0:1, :]
        zm = jnp.max(z, axis=1, keepdims=True)
        lse = jnp.log(jnp.sum(jnp.exp(z - zm), axis=1, keepdims=True)) + zm
        outr[...] = z - lse

    return pl.pallas_call(
        kfn,
        out_shape=jax.ShapeDtypeStruct((B, O), jnp.float32),
    )(tf, Wfc, b2)


def _cheb(t0, src, dst, w, lopk, W, bias, relu, xs=None, Ws=None):
    s1 = _lop4(lopk, t0, src, dst, w)
    s2 = _lop4(lopk, s1, src, dst, w)
    t2 = _axpy(s2, t0)
    s3 = _lop4(lopk, t2, src, dst, w)
    return _combine(t0, s1, t2, s3, W, bias, relu, xs=xs, Ws=Ws)


def _res(t, src, dst, w, lopa, lopb, Wa, ba, Wb, bb, Ws, bs):
    h = _cheb(t, src, dst, w, lopa, Wa, ba, True)
    return _cheb(h, src, dst, w, lopb, Wb, bb + bs, True, xs=t, Ws=Ws)


def kernel(x, edge_index1, edge_weight1, edge_index2, edge_weight2,
           edge_index3, edge_weight3, W0, b0, W1a, b1a, W1b, b1b, Ws1, bs1,
           W2a, b2a, W2b, b2b, Ws2, bs2, W3a, b3a, W3b, b3b, Ws3, bs3,
           Wfc, bfc):
    B, CIN, V1 = x.shape
    V2, V3 = V1 // 4, V1 // 16
    E1, E2, E3 = edge_index1.shape[1], edge_index2.shape[1], edge_index3.shape[1]
    nx3 = 6
    nx1_1 = 96
    nx1_2 = 48

    xb = jnp.transpose(x, (0, 2, 1))
    x16 = jnp.pad(xb, ((0, 0), (0, 0), (0, 16 - CIN)))
    W0p = jnp.pad(W0, ((0, 0), (0, 16 - CIN), (0, 0)))

    s1_, d1_, w1_ = edge_index1[0], edge_index1[1], edge_weight1
    s2_, d2_, w2_ = edge_index2[0], edge_index2[1], edge_weight2
    s3_, d3_, w3_ = edge_index3[0], edge_index3[1], edge_weight3

    H0, H1, H2, H3 = W0.shape[2], W1a.shape[2], W2a.shape[2], W3a.shape[2]
    lop1_16 = _make_lop(V1, 16, E1)
    lop1_32 = _make_lop(V1, H1, E1)
    lop2_32 = _make_lop(V2, H1, E2)
    lop2_64 = _make_lop(V2, H2, E2)
    lop3_64 = _make_lop(V3, H2, E3)
    lop3_128 = _make_lop(V3, H3, E3)

    out = _cheb(x16, s1_, d1_, w1_, lop1_16, W0p, b0, True)
    out = _res(out, s1_, d1_, w1_, lop1_16, lop1_32, W1a, b1a, W1b, b1b,
               Ws1, bs1)
    out = _pool(out, nx3, nx1_1, nx1_1)
    out = _res(out, s2_, d2_, w2_, lop2_32, lop2_64, W2a, b2a, W2b, b2b,
               Ws2, bs2)
    out = _pool(out, nx3, nx1_2, nx1_2)
    out = _res(out, s3_, d3_, w3_, lop3_64, lop3_128, W3a, b3a, W3b, b3b,
               Ws3, bs3)
    return _head(out, Wfc, bfc)
